# baseline (speedup 1.0000x reference)
import functools

import jax
import jax.numpy as jnp
from jax import lax
from jax.experimental import pallas as pl
from jax.experimental.pallas import tpu as pltpu
from jax.experimental.pallas import tpu_sc as plsc

B, S, D = 8, 2048, 256
N = 2048
PD = 20
PDP = 32
DOUT = D + D + PD + D + PD + PD
BINS = (0, 1, 2, 3, 4, 5, 7, 8, 9, 10, 15, 16, 31, 32, 63, 64)

NC, NS, L = 2, 16, 16
NW = NC * NS
NBLK = NW // B
SPW = N // NBLK
C = 32
NCHUNK = SPW // C

R = 256


def _cs_body(x_ref, o_ref, carry_ref):
    k = pl.program_id(1)

    @pl.when(k == 0)
    def _():
        carry_ref[...] = jnp.zeros_like(carry_ref)

    xb = x_ref[0]
    ri = lax.broadcasted_iota(jnp.int32, (R, R), 0)
    ci = lax.broadcasted_iota(jnp.int32, (R, R), 1)
    tril = (ri >= ci).astype(jnp.float32)
    cs = lax.dot(tril, xb, precision=lax.Precision.HIGHEST)
    cs = cs + carry_ref[...]
    o_ref[0] = cs
    carry_ref[...] = cs[R - 1:R, :]


_cumsum = pl.pallas_call(
    _cs_body,
    grid=(B, S // R),
    in_specs=[pl.BlockSpec((1, R, D), lambda b, k: (b, k, 0))],
    out_specs=pl.BlockSpec((1, R, D), lambda b, k: (b, k, 0)),
    out_shape=jax.ShapeDtypeStruct((B, S, D), jnp.float32),
    scratch_shapes=[pltpu.VMEM((1, D), jnp.float32)],
)


def _sc_body(cs_hbm, x_hbm, s0_hbm, s1_hbm, pt_hbm, pos_hbm, wid_hbm,
             out_hbm,
             s0_b, s1_b, g0_v, g1_v, pt0_v, pt1_v, em_v, invw_v, pt_v,
             cs0_b, cs1_b, x0_b, x1_b, p0_b, p1_b, w_b, out_buf, sem):
    cid = lax.axis_index("c")
    sid = lax.axis_index("s")
    wid = sid * NC + cid
    b = wid // NBLK
    n0 = (wid % NBLK) * SPW
    row0 = b * N + n0

    pltpu.sync_copy(pt_hbm, pt_v)

    def chunk(i, _):
        pltpu.sync_copy(s0_hbm.at[pl.ds(n0 + i * C, C)], s0_b)
        pltpu.sync_copy(s1_hbm.at[pl.ds(n0 + i * C, C)], s1_b)

        for g in range(C // L):
            s0 = s0_b[pl.ds(g * L, L)]
            s1 = s1_b[pl.ds(g * L, L)]
            w = s1 - s0 + 1
            invw_v[pl.ds(g * L, L)] = 1.0 / w.astype(jnp.float32)
            g0_v[pl.ds(g * L, L)] = s0 + b * S
            g1_v[pl.ds(g * L, L)] = s1 + b * S
            pt0_v[pl.ds(g * L, L)] = plsc.load_gather(pt_v, [s0])
            pt1_v[pl.ds(g * L, L)] = plsc.load_gather(pt_v, [s1])
            acc = jnp.zeros((L,), jnp.int32)
            for bv in BINS[1:]:
                acc += (w >= bv).astype(jnp.int32)
            em_v[pl.ds(g * L, L)] = acc

        d1 = pltpu.async_copy(cs_hbm.at[g1_v], cs1_b, sem)
        d2 = pltpu.async_copy(cs_hbm.at[g0_v], cs0_b, sem)
        d3 = pltpu.async_copy(x_hbm.at[g0_v], x0_b, sem)
        d4 = pltpu.async_copy(x_hbm.at[g1_v], x1_b, sem)
        d5 = pltpu.async_copy(pos_hbm.at[pt0_v], p0_b, sem)
        d6 = pltpu.async_copy(pos_hbm.at[pt1_v], p1_b, sem)
        d7 = pltpu.async_copy(wid_hbm.at[em_v], w_b, sem)
        d1.wait(); d2.wait(); d3.wait(); d4.wait(); d5.wait(); d6.wait()
        d7.wait()

        def row(r, _):
            iw = plsc.load_gather(invw_v, [jnp.full((L,), 0, jnp.int32) + r])
            for k in range(D // L):
                sl = pl.ds(k * L, L)
                x0v = x0_b[r, sl]
                seg = cs1_b[r, sl] - cs0_b[r, sl] + x0v
                out_buf[r, pl.ds(k * L, L)] = seg * iw
                out_buf[r, pl.ds(D + k * L, L)] = x0v
                out_buf[r, pl.ds(2 * D + PD + k * L, L)] = x1_b[r, sl]
            out_buf[r, pl.ds(2 * D, L)] = p0_b[r, pl.ds(0, L)]
            out_buf[r, pl.ds(2 * D + PD - L, L)] = p0_b[r, pl.ds(PD - L, L)]
            out_buf[r, pl.ds(3 * D + PD, L)] = p1_b[r, pl.ds(0, L)]
            out_buf[r, pl.ds(3 * D + 2 * PD - L, L)] = p1_b[r, pl.ds(PD - L, L)]
            out_buf[r, pl.ds(3 * D + 2 * PD, L)] = w_b[r, pl.ds(0, L)]
            out_buf[r, pl.ds(3 * D + 3 * PD - L, L)] = w_b[r, pl.ds(PD - L, L)]
            return 0

        lax.fori_loop(0, C, row, 0)

        rb = row0 + i * C
        o1 = pltpu.async_copy(out_buf, out_hbm.at[pl.ds(rb, C)], sem)
        o1.wait()
        return 0

    lax.fori_loop(0, NCHUNK, chunk, 0)


_SC_SCRATCH = [
    pltpu.VMEM((C,), jnp.int32),
    pltpu.VMEM((C,), jnp.int32),
    pltpu.VMEM((C,), jnp.int32),
    pltpu.VMEM((C,), jnp.int32),
    pltpu.VMEM((C,), jnp.int32),
    pltpu.VMEM((C,), jnp.int32),
    pltpu.VMEM((C,), jnp.int32),
    pltpu.VMEM((C,), jnp.float32),
    pltpu.VMEM((S,), jnp.int32),
    pltpu.VMEM((C, D), jnp.float32),
    pltpu.VMEM((C, D), jnp.float32),
    pltpu.VMEM((C, D), jnp.float32),
    pltpu.VMEM((C, D), jnp.float32),
    pltpu.VMEM((C, PDP), jnp.float32),
    pltpu.VMEM((C, PDP), jnp.float32),
    pltpu.VMEM((C, PDP), jnp.float32),
    pltpu.VMEM((C, DOUT), jnp.float32),
    pltpu.SemaphoreType.DMA,
]

_sc_kernel = functools.partial(
    pl.kernel,
    out_type=jax.ShapeDtypeStruct((B * N, DOUT), jnp.float32),
    mesh=plsc.VectorSubcoreMesh(core_axis_name="c", subcore_axis_name="s",
                                num_cores=NC, num_subcores=NS),
    compiler_params=pltpu.CompilerParams(use_tc_tiling_on_sc=False,
                                         needs_layout_passes=False),
    scratch_types=_SC_SCRATCH,
)(_sc_body)


@jax.jit
def kernel(x, spans_indices, span_pt_labels, width_table, pos_table):
    cs = _cumsum(x)
    xf = x.reshape(B * S, D)
    csf = cs.reshape(B * S, D)
    s0 = spans_indices[0, :, 0].astype(jnp.int32)
    s1 = spans_indices[0, :, 1].astype(jnp.int32)
    pt = span_pt_labels[0].astype(jnp.int32)
    pos_pad = jnp.pad(pos_table, ((0, 0), (0, PDP - PD)))
    wid_pad = jnp.pad(width_table, ((0, 0), (0, PDP - PD)))
    out = _sc_kernel(csf, xf, s0, s1, pt, pos_pad, wid_pad)
    return out.reshape(B, N, DOUT)
